# baseline (speedup 1.0000x reference)
import functools

import jax
import jax.numpy as jnp
from jax import lax
from jax.experimental import pallas as pl
from jax.experimental.pallas import tpu as pltpu
from jax.experimental.pallas import tpu_sc as plsc

_K = 32
_B = 16384
_NC = 2
_NS = 16
_NW = _NC * _NS
_BPW = _B // _NW
_L = 16


def _body(users_hbm, items_hbm, neg_hbm, p_hbm, q_hbm, pos_out, neg_out,
          idx_u, idx_i, idx_n, rows_u, rows_i, rows_n, pos_v, neg_v, sem):
    wid = lax.axis_index("s") * _NC + lax.axis_index("c")
    base = wid * _BPW

    pltpu.sync_copy(users_hbm.at[pl.ds(base, _BPW)], idx_u)
    pltpu.sync_copy(items_hbm.at[pl.ds(base, _BPW)], idx_i)
    pltpu.sync_copy(neg_hbm.at[pl.ds(base, _BPW)], idx_n)

    cu = pltpu.async_copy(p_hbm.at[idx_u], rows_u, sem)
    ci = pltpu.async_copy(q_hbm.at[idx_i], rows_i, sem)
    cn = pltpu.async_copy(q_hbm.at[idx_n], rows_n, sem)
    cu.wait()
    ci.wait()
    cn.wait()

    iota = lax.iota(jnp.int32, _L)
    zeros = jnp.zeros((_L,), jnp.float32)

    def group(g, carry):
        row = g * _L + iota
        acc_p = zeros
        acc_n = zeros
        for k in range(_K):
            col = jnp.full((_L,), k, jnp.int32)
            u = plsc.load_gather(rows_u, [row, col])
            qi = plsc.load_gather(rows_i, [row, col])
            qn = plsc.load_gather(rows_n, [row, col])
            acc_p = acc_p + u * qi
            acc_n = acc_n + u * qn
        pos_v[pl.ds(g * _L, _L)] = acc_p
        neg_v[pl.ds(g * _L, _L)] = acc_n
        return carry

    lax.fori_loop(0, _BPW // _L, group, 0)

    pltpu.sync_copy(pos_v, pos_out.at[pl.ds(base, _BPW)])
    pltpu.sync_copy(neg_v, neg_out.at[pl.ds(base, _BPW)])


@functools.partial(jax.jit, static_argnums=())
def _run(users, items, neg_items, p, q):
    mesh = plsc.VectorSubcoreMesh(core_axis_name="c", subcore_axis_name="s")
    f = pl.kernel(
        _body,
        mesh=mesh,
        out_type=(
            jax.ShapeDtypeStruct((_B,), jnp.float32),
            jax.ShapeDtypeStruct((_B,), jnp.float32),
        ),
        scratch_types=[
            pltpu.VMEM((_BPW,), jnp.int32),
            pltpu.VMEM((_BPW,), jnp.int32),
            pltpu.VMEM((_BPW,), jnp.int32),
            pltpu.VMEM((_BPW, _K), jnp.float32),
            pltpu.VMEM((_BPW, _K), jnp.float32),
            pltpu.VMEM((_BPW, _K), jnp.float32),
            pltpu.VMEM((_BPW,), jnp.float32),
            pltpu.VMEM((_BPW,), jnp.float32),
            pltpu.SemaphoreType.DMA,
        ],
        compiler_params=pltpu.CompilerParams(
            needs_layout_passes=False, use_tc_tiling_on_sc=False
        ),
    )
    return f(users, items, neg_items, p, q)


def kernel(users, items, neg_items, P, Q):
    users = users.astype(jnp.int32)
    items = items.astype(jnp.int32)
    neg_items = neg_items.astype(jnp.int32)
    return _run(users, items, neg_items, P, Q)
